# Initial kernel scaffold; baseline (speedup 1.0000x reference)
import jax
import jax.numpy as jnp
from jax.experimental import pallas as pl


def kernel(x, edge_index, edge_weights, W_in, b_in, g_in, be_in, bn1_g, bn1_b, Wl1, bl1, Wr1, br1, We1, att1, cb1, bn2_g, bn2_b, Wl2, bl2, Wr2, br2, We2, att2, cb2, W_out, b_out):
    raise NotImplementedError("write your pallas kernel here")


# speedup vs baseline: 6.7726x; 6.7726x over previous
import functools
import jax
import jax.numpy as jnp
from jax import lax
from jax.experimental import pallas as pl
from jax.experimental.pallas import tpu as pltpu
from jax.experimental.pallas import tpu_sc as plsc

N = 10000
E = 320000
C = 32
D_EDGE = 16
D_IN = 128
D_OUT = 128

SC_NC = 2
SC_NS = 16
NW = SC_NC * SC_NS
N_ACC = 10240
SLICE = N_ACC // SC_NS
CHUNK_ROWS = 3
K_CHUNK = CHUNK_ROWS * 128
N_CHUNKS = 28
ROWS_PER_W = N_CHUNKS * CHUNK_ROWS
IDX_ROWS = ROWS_PER_W * NW
E_PAD = IDX_ROWS * 128


def _leaky(v, s):
    return jnp.where(v >= 0, v, s * v)


def _bn_act(h, g, b):
    m = jnp.mean(h, axis=0)
    d = h - m
    v = jnp.mean(d * d, axis=0)
    hn = d / jnp.sqrt(v + 1e-5) * g + b
    return _leaky(hn, 0.01)


def _store_padded(ref, val):
    ref[...] = jnp.zeros((N_ACC, C), jnp.float32)
    ref[0:N, :] = val


def _pre_body(x_ref, win_ref, bin_ref, gin_ref, bein_ref, g1_ref, b1_ref,
              wl_ref, bl_ref, wr_ref, br_ref,
              h0_ref, xl_ref, xr_ref, st_ref):
    x = x_ref[...]
    h = jnp.dot(x, win_ref[...], preferred_element_type=jnp.float32) + bin_ref[...]
    h0 = _bn_act(h, gin_ref[...], bein_ref[...])
    h0_ref[...] = h0
    hh = _bn_act(h0, g1_ref[...], b1_ref[...])
    xl = jnp.dot(hh, wl_ref[...], preferred_element_type=jnp.float32) + bl_ref[...]
    xr = jnp.dot(hh, wr_ref[...], preferred_element_type=jnp.float32) + br_ref[...]
    _store_padded(xl_ref, xl)
    _store_padded(xr_ref, xr)
    st_ref[...] = jnp.stack([xl.max(0), xl.min(0), xr.max(0), xr.min(0)])


_pre_call = pl.pallas_call(
    _pre_body,
    out_shape=(
        jax.ShapeDtypeStruct((N, C), jnp.float32),
        jax.ShapeDtypeStruct((N_ACC, C), jnp.float32),
        jax.ShapeDtypeStruct((N_ACC, C), jnp.float32),
        jax.ShapeDtypeStruct((4, C), jnp.float32),
    ),
)


def _mid_body(h_ref, num_ref, den_ref, cb_ref, g2_ref, b2_ref,
              wl_ref, bl_ref, wr_ref, br_ref,
              h1_ref, xl_ref, xr_ref, st_ref):
    num = num_ref[0, 0:N, :] + num_ref[1, 0:N, :]
    den = den_ref[0, 0:N, :] + den_ref[1, 0:N, :]
    conv = jnp.where(den > 0, num / jnp.where(den > 0, den, 1.0), 0.0)
    h1 = h_ref[...] + conv + cb_ref[...]
    h1_ref[...] = h1
    hh = _bn_act(h1, g2_ref[...], b2_ref[...])
    xl = jnp.dot(hh, wl_ref[...], preferred_element_type=jnp.float32) + bl_ref[...]
    xr = jnp.dot(hh, wr_ref[...], preferred_element_type=jnp.float32) + br_ref[...]
    _store_padded(xl_ref, xl)
    _store_padded(xr_ref, xr)
    st_ref[...] = jnp.stack([xl.max(0), xl.min(0), xr.max(0), xr.min(0)])


_mid_call = pl.pallas_call(
    _mid_body,
    out_shape=(
        jax.ShapeDtypeStruct((N, C), jnp.float32),
        jax.ShapeDtypeStruct((N_ACC, C), jnp.float32),
        jax.ShapeDtypeStruct((N_ACC, C), jnp.float32),
        jax.ShapeDtypeStruct((4, C), jnp.float32),
    ),
)


def _out_body(h_ref, num_ref, den_ref, cb_ref, wout_ref, bout_ref, y_ref):
    num = num_ref[0, 0:N, :] + num_ref[1, 0:N, :]
    den = den_ref[0, 0:N, :] + den_ref[1, 0:N, :]
    conv = jnp.where(den > 0, num / jnp.where(den > 0, den, 1.0), 0.0)
    h2 = h_ref[...] + conv + cb_ref[...]
    y_ref[...] = jnp.dot(h2, wout_ref[...], preferred_element_type=jnp.float32) + bout_ref[...]


_out_call = pl.pallas_call(
    _out_body,
    out_shape=jax.ShapeDtypeStruct((N, D_OUT), jnp.float32),
)


EE_BLK = 4096
EE_GRID = E_PAD // EE_BLK


def _ee_body(ew_ref, we1_ref, we2_ref, e1_ref, e2_ref, st_ref):
    ew = ew_ref[...]
    e1 = jnp.dot(ew, we1_ref[...], preferred_element_type=jnp.float32)
    e2 = jnp.dot(ew, we2_ref[...], preferred_element_type=jnp.float32)
    e1_ref[...] = e1
    e2_ref[...] = e2
    st_ref[0] = jnp.stack([e1.max(0), e1.min(0), e2.max(0), e2.min(0)])


_ee_call = pl.pallas_call(
    _ee_body,
    grid=(EE_GRID,),
    in_specs=[
        pl.BlockSpec((EE_BLK, D_EDGE), lambda k: (k, 0)),
        pl.BlockSpec((D_EDGE, C), lambda k: (0, 0)),
        pl.BlockSpec((D_EDGE, C), lambda k: (0, 0)),
    ],
    out_specs=(
        pl.BlockSpec((EE_BLK, C), lambda k: (k, 0)),
        pl.BlockSpec((EE_BLK, C), lambda k: (k, 0)),
        pl.BlockSpec((1, 4, C), lambda k: (k, 0, 0)),
    ),
    out_shape=(
        jax.ShapeDtypeStruct((E_PAD, C), jnp.float32),
        jax.ShapeDtypeStruct((E_PAD, C), jnp.float32),
        jax.ShapeDtypeStruct((EE_GRID, 4, C), jnp.float32),
    ),
)


def _edge_body(xl_hbm, xr_hbm, ee_hbm, idx_hbm, attr_hbm, bsh_hbm,
               num_out, den_out,
               idx0, idx1, xl0, xl1, xr0, xr1, ee0, ee1,
               w_v, e_v, lg_v, att_v, bsh_v, num_sh, den_sh,
               gsem0, gsem1, ssem):
    c_id = lax.axis_index("c")
    s_id = lax.axis_index("s")
    wid = s_id * SC_NC + c_id

    def issue(k, idxb, xlb, xrb, eeb, sem):
        cg = wid * N_CHUNKS + k
        pltpu.sync_copy(idx_hbm.at[cg], idxb)
        pltpu.async_copy(xl_hbm.at[idxb.at[0]], xlb, sem)
        pltpu.async_copy(xr_hbm.at[idxb.at[1]], xrb, sem)
        pltpu.async_copy(ee_hbm.at[pl.ds(cg * K_CHUNK, K_CHUNK), :], eeb, sem)

    def drain(xlb, xrb, eeb, sem):
        pltpu.make_async_copy(xl_hbm.at[pl.ds(0, K_CHUNK), :], xlb, sem).wait()
        pltpu.make_async_copy(xr_hbm.at[pl.ds(0, K_CHUNK), :], xrb, sem).wait()
        pltpu.make_async_copy(ee_hbm.at[pl.ds(0, K_CHUNK), :], eeb, sem).wait()

    pltpu.sync_copy(attr_hbm, att_v)
    pltpu.sync_copy(bsh_hbm, bsh_v)

    def zero2_body(r, _):
        ee0[r, 0:16] = jnp.zeros((16,), jnp.float32)
        ee0[r, 16:32] = jnp.zeros((16,), jnp.float32)
        return 0
    lax.fori_loop(0, K_CHUNK, zero2_body, 0)

    def zero1_body(r, _):
        e_v[pl.ds(r * 16, 16)] = jnp.zeros((16,), jnp.float32)
        return 0
    lax.fori_loop(0, K_CHUNK // 16, zero1_body, 0)

    base = s_id * SLICE
    pltpu.sync_copy(ee0, num_sh.at[pl.ds(base, K_CHUNK), :])
    pltpu.sync_copy(ee0.at[pl.ds(0, SLICE - K_CHUNK), :],
                    num_sh.at[pl.ds(base + K_CHUNK, SLICE - K_CHUNK), :])
    pltpu.sync_copy(e_v, den_sh.at[pl.ds(base, K_CHUNK)])
    pltpu.sync_copy(e_v.at[pl.ds(0, SLICE - K_CHUNK)],
                    den_sh.at[pl.ds(base + K_CHUNK, SLICE - K_CHUNK)])
    issue(0, idx0, xl0, xr0, ee0, gsem0)
    plsc.subcore_barrier()

    bsh = bsh_v[...]

    CB = 8

    def compute(xlb, xrb, eeb):
        ngroups = CHUNK_ROWS * 8
        for cb in range(C // CB):
            att_s = [att_v[cb * CB + i, :] for i in range(CB)]

            @plsc.parallel_loop(0, ngroups, 1, unroll=2)
            def lbody(g, cb=cb, att_s=att_s):
                zz = jnp.zeros((16,), jnp.int32)
                f0 = lax.iota(jnp.int32, 16) * C + (g * (16 * C) + cb * CB)
                if cb == 0:
                    acc = jnp.zeros((16,), jnp.float32)
                else:
                    acc = lg_v[pl.ds(g * 16, 16)]
                for i in range(CB):
                    f = f0 + i
                    xlg = plsc.load_gather(xlb, [zz, f])
                    xrg = plsc.load_gather(xrb, [zz, f])
                    eeg = plsc.load_gather(eeb, [zz, f])
                    m = xlg + xrg + eeg
                    acc = acc + jnp.maximum(m, 0.2 * m) * att_s[i]
                lg_v[pl.ds(g * 16, 16)] = acc

        @plsc.parallel_loop(0, ngroups, 1, unroll=2)
        def ebody(g):
            acc = lg_v[pl.ds(g * 16, 16)]
            e_v[pl.ds(g * 16, 16)] = jnp.exp(jnp.maximum(acc - bsh, -80.0))

        for cb in range(C // CB):
            @plsc.parallel_loop(0, ngroups, 1, unroll=2)
            def wbody(g, cb=cb):
                zz = jnp.zeros((16,), jnp.int32)
                f0 = lax.iota(jnp.int32, 16) * C + (g * (16 * C) + cb * CB)
                e16 = e_v[pl.ds(g * 16, 16)]
                for i in range(CB):
                    f = f0 + i
                    xlg = plsc.load_gather(xlb, [zz, f])
                    plsc.store_scatter(w_v, [zz, f], xlg * e16)

    def scatter_issue(idxb):
        pltpu.async_copy(w_v, num_sh.at[idxb.at[1]], ssem, add=True)
        pltpu.async_copy(e_v, den_sh.at[idxb.at[1]], ssem, add=True)

    def scatter_drain_all():
        pltpu.make_async_copy(xl_hbm.at[pl.ds(0, K_CHUNK), :], w_v, ssem).wait()
        pltpu.make_async_copy(den_out.at[0].at[pl.ds(0, K_CHUNK)], e_v, ssem).wait()

    def pair_body(k2, _):
        k0 = 2 * k2
        drain(xl0, xr0, ee0, gsem0)

        @pl.when(k2 > 0)
        def _():
            scatter_drain_all()

        issue(k0 + 1, idx1, xl1, xr1, ee1, gsem1)
        compute(xl0, xr0, ee0)
        scatter_issue(idx0)
        drain(xl1, xr1, ee1, gsem1)
        scatter_drain_all()

        @pl.when(k2 < N_CHUNKS // 2 - 1)
        def _():
            issue(k0 + 2, idx0, xl0, xr0, ee0, gsem0)

        compute(xl1, xr1, ee1)
        scatter_issue(idx1)
        return 0

    lax.fori_loop(0, N_CHUNKS // 2, pair_body, 0)
    scatter_drain_all()
    plsc.subcore_barrier()

    pltpu.sync_copy(num_sh.at[pl.ds(s_id * SLICE, SLICE), :],
                    num_out.at[c_id].at[pl.ds(s_id * SLICE, SLICE), :])
    pltpu.sync_copy(den_sh.at[pl.ds(s_id * SLICE, SLICE)],
                    den_out.at[c_id].at[pl.ds(s_id * SLICE, SLICE)])


_edge_call = pl.kernel(
    _edge_body,
    out_type=(
        jax.ShapeDtypeStruct((SC_NC, N_ACC, C), jnp.float32),
        jax.ShapeDtypeStruct((SC_NC, N_ACC), jnp.float32),
    ),
    mesh=plsc.VectorSubcoreMesh(core_axis_name="c", subcore_axis_name="s",
                                num_cores=SC_NC, num_subcores=SC_NS),
    scratch_types=[
        pltpu.VMEM((2, K_CHUNK), jnp.int32),
        pltpu.VMEM((2, K_CHUNK), jnp.int32),
        pltpu.VMEM((K_CHUNK, C), jnp.float32),
        pltpu.VMEM((K_CHUNK, C), jnp.float32),
        pltpu.VMEM((K_CHUNK, C), jnp.float32),
        pltpu.VMEM((K_CHUNK, C), jnp.float32),
        pltpu.VMEM((K_CHUNK, C), jnp.float32),
        pltpu.VMEM((K_CHUNK, C), jnp.float32),
        pltpu.VMEM((K_CHUNK, C), jnp.float32),
        pltpu.VMEM((K_CHUNK,), jnp.float32),
        pltpu.VMEM((K_CHUNK,), jnp.float32),
        pltpu.VMEM((C, 16), jnp.float32),
        pltpu.VMEM((16,), jnp.float32),
        pltpu.VMEM_SHARED((N_ACC, C), jnp.float32),
        pltpu.VMEM_SHARED((N_ACC,), jnp.float32),
        pltpu.SemaphoreType.DMA,
        pltpu.SemaphoreType.DMA,
        pltpu.SemaphoreType.DMA,
    ],
    compiler_params=pltpu.CompilerParams(needs_layout_passes=False,
                                         use_tc_tiling_on_sc=False),
    name="gat_edge_sc",
)


def _logit_bound(att, st_xlxr, st_ee):
    hi = st_xlxr[0] + st_xlxr[2] + st_ee[0]
    lo = st_xlxr[1] + st_xlxr[3] + st_ee[1]
    return jnp.sum(jnp.maximum(att * _leaky(hi, 0.2), att * _leaky(lo, 0.2)))


def kernel(x, edge_index, edge_weights, W_in, b_in, g_in, be_in,
           bn1_g, bn1_b, Wl1, bl1, Wr1, br1, We1, att1, cb1,
           bn2_g, bn2_b, Wl2, bl2, Wr2, br2, We2, att2, cb2,
           W_out, b_out):
    src = edge_index[0]
    dst = edge_index[1]
    pad = E_PAD - E
    pad_idx = (N + (jnp.arange(pad, dtype=jnp.int32) % 64)).astype(jnp.int32)
    src_p = jnp.concatenate([src.astype(jnp.int32), pad_idx]).reshape(-1, K_CHUNK)
    dst_p = jnp.concatenate([dst.astype(jnp.int32), pad_idx]).reshape(-1, K_CHUNK)
    idx_p = jnp.stack([src_p, dst_p], axis=1)
    ew_p = jnp.pad(edge_weights, ((0, pad), (0, 0)))

    h0, xl1, xr1, st1 = _pre_call(x, W_in, b_in, g_in, be_in, bn1_g, bn1_b,
                                  Wl1, bl1, Wr1, br1)
    ee1, ee2, ee_st = _ee_call(ew_p, We1, We2)
    ee_max = ee_st.max(0)
    ee_min = ee_st.min(0)

    b1v = _logit_bound(att1, st1, (ee_max[0], ee_min[1]))
    att1_rep = jnp.broadcast_to(att1[:, None], (C, 16))
    bsh1 = jnp.full((16,), b1v, jnp.float32)
    num1, den1 = _edge_call(xl1, xr1, ee1, idx_p, att1_rep, bsh1)

    h1, xl2, xr2, st2 = _mid_call(h0, num1, den1[:, :, None], cb1, bn2_g, bn2_b,
                                  Wl2, bl2, Wr2, br2)

    b2v = _logit_bound(att2, st2, (ee_max[2], ee_min[3]))
    att2_rep = jnp.broadcast_to(att2[:, None], (C, 16))
    bsh2 = jnp.full((16,), b2v, jnp.float32)
    num2, den2 = _edge_call(xl2, xr2, ee2, idx_p, att2_rep, bsh2)

    return _out_call(h1, num2, den2[:, :, None], cb2, W_out, b_out)
